# baseline (speedup 1.0000x reference)
import jax
import jax.numpy as jnp
from jax import lax
from jax.experimental import pallas as pl
from jax.experimental.pallas import tpu as pltpu
from jax.experimental.pallas import tpu_sc as plsc

DIM = 32
PACK = 4
LINE = DIM * PACK
NC = 2
NS = 16
NW = NC * NS
GRP = 128
LANES = 16


def _body(job_idx, geek_idx, job_w, geek_w, out_hbm,
          jidx_v, gidx_v, jline_v, gline_v,
          jr0, gr0, jr1, gr1, out_v, sem0, sem1):
    n_per_w = jidx_v.shape[0]
    n_grp = n_per_w // GRP
    wid = lax.axis_index("s") * NC + lax.axis_index("c")
    base = wid * n_per_w
    lanes = lax.iota(jnp.int32, LANES)

    pltpu.sync_copy(job_idx.at[pl.ds(base, n_per_w)], jidx_v)
    pltpu.sync_copy(geek_idx.at[pl.ds(base, n_per_w)], gidx_v)

    def shift(i, carry):
        s = pl.ds(i * LANES, LANES)
        jline_v[s] = lax.shift_right_logical(jidx_v[s], 2)
        gline_v[s] = lax.shift_right_logical(gidx_v[s], 2)
        return carry
    lax.fori_loop(0, n_per_w // LANES, shift, 0)

    def _sqrt(s):
        i = plsc.bitcast(s, jnp.int32)
        y = plsc.bitcast(
            jnp.int32(0x5F3759DF) - lax.shift_right_logical(i, 1),
            jnp.float32)
        for _ in range(3):
            y = y * (1.5 - 0.5 * s * y * y)
        return s * y

    def fire(g, jr, gr, sem):
        pltpu.async_copy(job_w.at[jline_v.at[pl.ds(g * GRP, GRP)]], jr, sem)
        pltpu.async_copy(geek_w.at[gline_v.at[pl.ds(g * GRP, GRP)]], gr, sem)

    def drain(jr, gr, sem):
        pltpu.make_async_copy(job_w.at[pl.ds(0, GRP)], jr, sem).wait()
        pltpu.make_async_copy(geek_w.at[pl.ds(0, GRP)], gr, sem).wait()

    def compute(g, jr, gr):
        def block(t, carry):
            s = pl.ds(g * GRP + t * LANES, LANES)
            jcol = lax.shift_left(jidx_v[s] & 3, 5)
            gcol = lax.shift_left(gidx_v[s] & 3, 5)
            row = lanes + t * LANES
            acc = [jnp.zeros((LANES,), jnp.float32) for _ in range(4)]
            for d in range(DIM):
                rot = (lanes + d) & (DIM - 1)
                jv = plsc.load_gather(jr, [row, jcol + rot])
                gv = plsc.load_gather(gr, [row, gcol + rot])
                diff = jv - gv
                acc[d % 4] = acc[d % 4] + diff * diff
            ssum = (acc[0] + acc[1]) + (acc[2] + acc[3])
            out_v[pl.ds(g * GRP + t * LANES, LANES)] = _sqrt(ssum)
            return carry
        lax.fori_loop(0, GRP // LANES, block, 0)

    fire(0, jr0, gr0, sem0)

    def pair(i, carry):
        fire(2 * i + 1, jr1, gr1, sem1)
        drain(jr0, gr0, sem0)
        compute(2 * i, jr0, gr0)

        @pl.when(i < n_grp // 2 - 1)
        def _():
            fire(2 * i + 2, jr0, gr0, sem0)

        drain(jr1, gr1, sem1)
        compute(2 * i + 1, jr1, gr1)
        return carry

    lax.fori_loop(0, n_grp // 2, pair, 0)
    pltpu.sync_copy(out_v, out_hbm.at[pl.ds(base, n_per_w)])


@jax.jit
def _cml_norm(job_idx, geek_idx, job_w, geek_w):
    n = job_idx.shape[0]
    n_per_w = n // NW
    mesh = plsc.VectorSubcoreMesh(core_axis_name="c", subcore_axis_name="s")
    rows_t = pltpu.VMEM((GRP, LINE), jnp.float32)
    idx_t = pltpu.VMEM((n_per_w,), jnp.int32)
    return pl.kernel(
        _body,
        out_type=jax.ShapeDtypeStruct((n,), jnp.float32),
        mesh=mesh,
        compiler_params=pltpu.CompilerParams(
            needs_layout_passes=False, use_tc_tiling_on_sc=True),
        scratch_types=[
            idx_t, idx_t, idx_t, idx_t,
            rows_t, rows_t, rows_t, rows_t,
            pltpu.VMEM((n_per_w,), jnp.float32),
            pltpu.SemaphoreType.DMA,
            pltpu.SemaphoreType.DMA,
        ],
    )(job_idx, geek_idx, job_w, geek_w)


def kernel(job, geek, job_emb_w, geek_emb_w):
    B, L = job.shape
    jf = job.astype(jnp.int32).reshape(-1)
    gf = geek.astype(jnp.int32).reshape(-1)
    jw = job_emb_w.reshape(-1, LINE)
    gw = geek_emb_w.reshape(-1, LINE)
    out = _cml_norm(jf, gf, jw, gw)
    return out.reshape(B, L)


# speedup vs baseline: 1.1692x; 1.1692x over previous
import jax
import jax.numpy as jnp
from jax import lax
from jax.experimental import pallas as pl
from jax.experimental.pallas import tpu as pltpu
from jax.experimental.pallas import tpu_sc as plsc

DIM = 32
PACK = 4
LINE = DIM * PACK
NC = 2
NS = 16
NW = NC * NS
GRP = 128
LANES = 16
CHUNK = 512
SUB = CHUNK // GRP


def _transpose_body(jt, gt, jtail, gtail, jp, gp, inb, outb0, outb1,
                    isem, osem0, osem1):
    n = jt.shape[1]
    n_chunks = n // CHUNK
    n_tail_lines = (n - n_chunks * CHUNK) * DIM // LINE
    n_iters = (n_chunks + NW - 1) // NW
    wid = lax.axis_index("s") * NC + lax.axis_index("c")

    lanes = lax.iota(jnp.int32, LANES)
    lane_q = lax.shift_right_logical(lanes, 2)
    lane_p = lax.shift_left(lanes & 3, 5)

    def fill(src, chunk):
        for q in range(SUB):
            pltpu.async_copy(
                src.at[pl.ds(0, DIM), pl.ds(chunk * CHUNK + q * GRP, GRP)],
                inb.at[pl.ds(q * DIM, DIM)], isem)

    def drain_fill():
        pltpu.make_async_copy(
            jt.at[pl.ds(0, DIM), pl.ds(0, GRP)],
            inb.at[pl.ds(0, DIM)], isem).wait()

    def transpose_to(outb):
        def block(bi, carry):
            q, s = bi // 8, bi % 8
            col = s * LANES + lanes
            line = q * DIM + s * 4 + lane_q
            for d0 in (0, LANES):
                for k in range(LANES):
                    dvec = d0 + ((lanes + k) & (LANES - 1))
                    v = plsc.load_gather(inb, [q * DIM + dvec, col])
                    plsc.store_scatter(outb, [line, lane_p + dvec], v)
            return carry
        lax.fori_loop(0, SUB * 8, block, 0)

    def flush(outb, dst, chunk, sem):
        pltpu.async_copy(outb, dst.at[pl.ds(chunk * (CHUNK // PACK),
                                            CHUNK // PACK)], sem)

    def step(i, carry):
        chunk = wid + NW * i

        @pl.when(chunk < n_chunks)
        def _():
            fill(jt, chunk)
            for _ in range(SUB):
                drain_fill()
            @pl.when(i > 0)
            def _():
                pltpu.make_async_copy(
                    outb0, jp.at[pl.ds(0, CHUNK // PACK)], osem0).wait()
            transpose_to(outb0)
            flush(outb0, jp, chunk, osem0)

            fill(gt, chunk)
            for _ in range(SUB):
                drain_fill()
            @pl.when(i > 0)
            def _():
                pltpu.make_async_copy(
                    outb1, gp.at[pl.ds(0, CHUNK // PACK)], osem1).wait()
            transpose_to(outb1)
            flush(outb1, gp, chunk, osem1)
        return carry

    lax.fori_loop(0, n_iters, step, 0)

    pltpu.make_async_copy(outb0, jp.at[pl.ds(0, CHUNK // PACK)],
                          osem0).wait()
    pltpu.make_async_copy(outb1, gp.at[pl.ds(0, CHUNK // PACK)],
                          osem1).wait()

    @pl.when(wid == NW - 1)
    def _():
        pltpu.sync_copy(jtail, inb.at[pl.ds(0, n_tail_lines)])
        pltpu.sync_copy(inb.at[pl.ds(0, n_tail_lines)],
                        jp.at[pl.ds(n_chunks * (CHUNK // PACK),
                                    n_tail_lines)])
        pltpu.sync_copy(gtail, inb.at[pl.ds(0, n_tail_lines)])
        pltpu.sync_copy(inb.at[pl.ds(0, n_tail_lines)],
                        gp.at[pl.ds(n_chunks * (CHUNK // PACK),
                                    n_tail_lines)])


def _cml_body(job_idx, geek_idx, job_w, geek_w, out_hbm,
              jidx_v, gidx_v, jline_v, gline_v,
              jr0, gr0, jr1, gr1, out_v, sem0, sem1):
    n_per_w = jidx_v.shape[0]
    n_grp = n_per_w // GRP
    wid = lax.axis_index("s") * NC + lax.axis_index("c")
    base = wid * n_per_w
    lanes = lax.iota(jnp.int32, LANES)

    pltpu.sync_copy(job_idx.at[pl.ds(base, n_per_w)], jidx_v)
    pltpu.sync_copy(geek_idx.at[pl.ds(base, n_per_w)], gidx_v)

    def shift(i, carry):
        s = pl.ds(i * LANES, LANES)
        jline_v[s] = lax.shift_right_logical(jidx_v[s], 2)
        gline_v[s] = lax.shift_right_logical(gidx_v[s], 2)
        return carry
    lax.fori_loop(0, n_per_w // LANES, shift, 0)

    def _sqrt(s):
        i = plsc.bitcast(s, jnp.int32)
        y = plsc.bitcast(
            jnp.int32(0x5F3759DF) - lax.shift_right_logical(i, 1),
            jnp.float32)
        for _ in range(3):
            y = y * (1.5 - 0.5 * s * y * y)
        return s * y

    def fire(g, jr, gr, sem):
        pltpu.async_copy(job_w.at[jline_v.at[pl.ds(g * GRP, GRP)]], jr, sem)
        pltpu.async_copy(geek_w.at[gline_v.at[pl.ds(g * GRP, GRP)]], gr, sem)

    def drain(jr, gr, sem):
        pltpu.make_async_copy(job_w.at[pl.ds(0, GRP)], jr, sem).wait()
        pltpu.make_async_copy(geek_w.at[pl.ds(0, GRP)], gr, sem).wait()

    def compute(g, jr, gr):
        def block(t, carry):
            s = pl.ds(g * GRP + t * LANES, LANES)
            jcol = lax.shift_left(jidx_v[s] & 3, 5)
            gcol = lax.shift_left(gidx_v[s] & 3, 5)
            row = lanes + t * LANES
            acc = [jnp.zeros((LANES,), jnp.float32) for _ in range(4)]
            for d in range(DIM):
                rot = (lanes + d) & (DIM - 1)
                jv = plsc.load_gather(jr, [row, jcol + rot])
                gv = plsc.load_gather(gr, [row, gcol + rot])
                diff = jv - gv
                acc[d % 4] = acc[d % 4] + diff * diff
            ssum = (acc[0] + acc[1]) + (acc[2] + acc[3])
            out_v[pl.ds(g * GRP + t * LANES, LANES)] = _sqrt(ssum)
            return carry
        lax.fori_loop(0, GRP // LANES, block, 0)

    fire(0, jr0, gr0, sem0)

    def pair(i, carry):
        fire(2 * i + 1, jr1, gr1, sem1)
        drain(jr0, gr0, sem0)
        compute(2 * i, jr0, gr0)

        @pl.when(i < n_grp // 2 - 1)
        def _():
            fire(2 * i + 2, jr0, gr0, sem0)

        drain(jr1, gr1, sem1)
        compute(2 * i + 1, jr1, gr1)
        return carry

    lax.fori_loop(0, n_grp // 2, pair, 0)
    pltpu.sync_copy(out_v, out_hbm.at[pl.ds(base, n_per_w)])


@jax.jit
def _cml(job_idx, geek_idx, job_emb_w, geek_emb_w):
    n_rows = job_emb_w.shape[0]
    n_lines = n_rows * DIM // LINE
    n_main = (n_rows // CHUNK) * CHUNK
    mesh = plsc.VectorSubcoreMesh(core_axis_name="c", subcore_axis_name="s")
    compact = pltpu.CompilerParams(
        needs_layout_passes=False, use_tc_tiling_on_sc=True)

    jt = jnp.swapaxes(job_emb_w, 0, 1)
    gt = jnp.swapaxes(geek_emb_w, 0, 1)
    jtail = job_emb_w[n_main:].reshape(-1, LINE)
    gtail = geek_emb_w[n_main:].reshape(-1, LINE)

    packed_t = jax.ShapeDtypeStruct((n_lines, LINE), jnp.float32)
    buf_t = pltpu.VMEM((CHUNK // PACK, LINE), jnp.float32)
    jp, gp = pl.kernel(
        _transpose_body,
        out_type=(packed_t, packed_t),
        mesh=mesh,
        compiler_params=compact,
        scratch_types=[
            buf_t, buf_t, buf_t,
            pltpu.SemaphoreType.DMA,
            pltpu.SemaphoreType.DMA,
            pltpu.SemaphoreType.DMA,
        ],
    )(jt, gt, jtail, gtail)

    n = job_idx.shape[0]
    n_per_w = n // NW
    rows_t = pltpu.VMEM((GRP, LINE), jnp.float32)
    idx_t = pltpu.VMEM((n_per_w,), jnp.int32)
    return pl.kernel(
        _cml_body,
        out_type=jax.ShapeDtypeStruct((n,), jnp.float32),
        mesh=mesh,
        compiler_params=compact,
        scratch_types=[
            idx_t, idx_t, idx_t, idx_t,
            rows_t, rows_t, rows_t, rows_t,
            pltpu.VMEM((n_per_w,), jnp.float32),
            pltpu.SemaphoreType.DMA,
            pltpu.SemaphoreType.DMA,
        ],
    )(job_idx, geek_idx, jp, gp)


def kernel(job, geek, job_emb_w, geek_emb_w):
    B, L = job.shape
    jf = job.astype(jnp.int32).reshape(-1)
    gf = geek.astype(jnp.int32).reshape(-1)
    out = _cml(jf, gf, job_emb_w, geek_emb_w)
    return out.reshape(B, L)


# speedup vs baseline: 1.4786x; 1.2646x over previous
import jax
import jax.numpy as jnp
from jax import lax
from jax.experimental import pallas as pl
from jax.experimental.pallas import tpu as pltpu
from jax.experimental.pallas import tpu_sc as plsc

DIM = 32
PACK = 4
LINE = DIM * PACK
NC = 2
NS = 16
NW = NC * NS
GRP = 128
LANES = 16
CHUNK = 512
SUB = CHUNK // GRP


def _transpose_body(jt, gt, jtail, gtail, jp, gp, inb0, inb1, outb0, outb1,
                    isem0, isem1, osem0, osem1):
    n = jt.shape[1]
    n_chunks = n // CHUNK
    n_tail_lines = (n - n_chunks * CHUNK) * DIM // LINE
    n_iters = (n_chunks + NW - 1) // NW
    wid = lax.axis_index("s") * NC + lax.axis_index("c")

    lanes = lax.iota(jnp.int32, LANES)
    lane_q = lax.shift_right_logical(lanes, 2)
    lane_p = lax.shift_left(lanes & 3, 5)

    def fill(src, chunk, inb, isem):
        for q in range(SUB):
            pltpu.async_copy(
                src.at[pl.ds(0, DIM), pl.ds(chunk * CHUNK + q * GRP, GRP)],
                inb.at[pl.ds(q * DIM, DIM)], isem)

    def drain_fill(inb, isem):
        for _ in range(SUB):
            pltpu.make_async_copy(
                jt.at[pl.ds(0, DIM), pl.ds(0, GRP)],
                inb.at[pl.ds(0, DIM)], isem).wait()

    def transpose_to(inb, outb):
        def block(bi, carry):
            q, s = bi // 8, bi % 8
            col = s * LANES + lanes
            line = q * DIM + s * 4 + lane_q
            for d0 in (0, LANES):
                for k in range(LANES):
                    dvec = d0 + ((lanes + k) & (LANES - 1))
                    v = plsc.load_gather(inb, [q * DIM + dvec, col])
                    plsc.store_scatter(outb, [line, lane_p + dvec], v)
            return carry
        lax.fori_loop(0, SUB * 8, block, 0)

    def flush(outb, dst, chunk, sem):
        pltpu.async_copy(outb, dst.at[pl.ds(chunk * (CHUNK // PACK),
                                            CHUNK // PACK)], sem)

    @pl.when(wid < n_chunks)
    def _():
        fill(jt, wid, inb0, isem0)

    def step(i, carry):
        chunk = wid + NW * i

        @pl.when(chunk < n_chunks)
        def _():
            fill(gt, chunk, inb1, isem1)
            drain_fill(inb0, isem0)
            @pl.when(i > 0)
            def _():
                pltpu.make_async_copy(
                    outb0, jp.at[pl.ds(0, CHUNK // PACK)], osem0).wait()
            transpose_to(inb0, outb0)
            flush(outb0, jp, chunk, osem0)

            @pl.when(chunk + NW < n_chunks)
            def _():
                fill(jt, chunk + NW, inb0, isem0)
            drain_fill(inb1, isem1)
            @pl.when(i > 0)
            def _():
                pltpu.make_async_copy(
                    outb1, gp.at[pl.ds(0, CHUNK // PACK)], osem1).wait()
            transpose_to(inb1, outb1)
            flush(outb1, gp, chunk, osem1)
        return carry

    lax.fori_loop(0, n_iters, step, 0)

    pltpu.make_async_copy(outb0, jp.at[pl.ds(0, CHUNK // PACK)],
                          osem0).wait()
    pltpu.make_async_copy(outb1, gp.at[pl.ds(0, CHUNK // PACK)],
                          osem1).wait()

    @pl.when(wid == NW - 1)
    def _():
        pltpu.sync_copy(jtail, inb0.at[pl.ds(0, n_tail_lines)])
        pltpu.sync_copy(inb0.at[pl.ds(0, n_tail_lines)],
                        jp.at[pl.ds(n_chunks * (CHUNK // PACK),
                                    n_tail_lines)])
        pltpu.sync_copy(gtail, inb0.at[pl.ds(0, n_tail_lines)])
        pltpu.sync_copy(inb0.at[pl.ds(0, n_tail_lines)],
                        gp.at[pl.ds(n_chunks * (CHUNK // PACK),
                                    n_tail_lines)])


def _cml_body(job_idx, geek_idx, job_w, geek_w, out_hbm,
              jidx_v, gidx_v, jline_v, gline_v,
              jr0, gr0, jr1, gr1, out_v, sem0, sem1):
    n_per_w = jidx_v.shape[0]
    n_grp = n_per_w // GRP
    wid = lax.axis_index("s") * NC + lax.axis_index("c")
    base = wid * n_per_w
    lanes = lax.iota(jnp.int32, LANES)

    pltpu.sync_copy(job_idx.at[pl.ds(base, n_per_w)], jidx_v)
    pltpu.sync_copy(geek_idx.at[pl.ds(base, n_per_w)], gidx_v)

    def shift(i, carry):
        s = pl.ds(i * LANES, LANES)
        jline_v[s] = lax.shift_right_logical(jidx_v[s], 2)
        gline_v[s] = lax.shift_right_logical(gidx_v[s], 2)
        return carry
    lax.fori_loop(0, n_per_w // LANES, shift, 0)

    def _sqrt(s):
        i = plsc.bitcast(s, jnp.int32)
        y = plsc.bitcast(
            jnp.int32(0x5F3759DF) - lax.shift_right_logical(i, 1),
            jnp.float32)
        for _ in range(3):
            y = y * (1.5 - 0.5 * s * y * y)
        return s * y

    def fire(g, jr, gr, sem):
        pltpu.async_copy(job_w.at[jline_v.at[pl.ds(g * GRP, GRP)]], jr, sem)
        pltpu.async_copy(geek_w.at[gline_v.at[pl.ds(g * GRP, GRP)]], gr, sem)

    def drain(jr, gr, sem):
        pltpu.make_async_copy(job_w.at[pl.ds(0, GRP)], jr, sem).wait()
        pltpu.make_async_copy(geek_w.at[pl.ds(0, GRP)], gr, sem).wait()

    def compute(g, jr, gr):
        def block(t, carry):
            s = pl.ds(g * GRP + t * LANES, LANES)
            jcol = lax.shift_left(jidx_v[s] & 3, 5)
            gcol = lax.shift_left(gidx_v[s] & 3, 5)
            row = lanes + t * LANES
            acc = [jnp.zeros((LANES,), jnp.float32) for _ in range(4)]
            for d in range(DIM):
                rot = (lanes + d) & (DIM - 1)
                jv = plsc.load_gather(jr, [row, jcol + rot])
                gv = plsc.load_gather(gr, [row, gcol + rot])
                diff = jv - gv
                acc[d % 4] = acc[d % 4] + diff * diff
            ssum = (acc[0] + acc[1]) + (acc[2] + acc[3])
            out_v[pl.ds(g * GRP + t * LANES, LANES)] = _sqrt(ssum)
            return carry
        lax.fori_loop(0, GRP // LANES, block, 0)

    fire(0, jr0, gr0, sem0)

    def pair(i, carry):
        fire(2 * i + 1, jr1, gr1, sem1)
        drain(jr0, gr0, sem0)
        compute(2 * i, jr0, gr0)

        @pl.when(i < n_grp // 2 - 1)
        def _():
            fire(2 * i + 2, jr0, gr0, sem0)

        drain(jr1, gr1, sem1)
        compute(2 * i + 1, jr1, gr1)
        return carry

    lax.fori_loop(0, n_grp // 2, pair, 0)
    pltpu.sync_copy(out_v, out_hbm.at[pl.ds(base, n_per_w)])


@jax.jit
def _cml(job_idx, geek_idx, job_emb_w, geek_emb_w):
    n_rows = job_emb_w.shape[0]
    n_lines = n_rows * DIM // LINE
    n_main = (n_rows // CHUNK) * CHUNK
    mesh = plsc.VectorSubcoreMesh(core_axis_name="c", subcore_axis_name="s")
    compact = pltpu.CompilerParams(
        needs_layout_passes=False, use_tc_tiling_on_sc=True)

    jt = jnp.swapaxes(job_emb_w, 0, 1)
    gt = jnp.swapaxes(geek_emb_w, 0, 1)
    jtail = job_emb_w[n_main:].reshape(-1, LINE)
    gtail = geek_emb_w[n_main:].reshape(-1, LINE)

    packed_t = jax.ShapeDtypeStruct((n_lines, LINE), jnp.float32)
    buf_t = pltpu.VMEM((CHUNK // PACK, LINE), jnp.float32)
    jp, gp = pl.kernel(
        _transpose_body,
        out_type=(packed_t, packed_t),
        mesh=mesh,
        compiler_params=compact,
        scratch_types=[
            buf_t, buf_t, buf_t, buf_t,
            pltpu.SemaphoreType.DMA,
            pltpu.SemaphoreType.DMA,
            pltpu.SemaphoreType.DMA,
            pltpu.SemaphoreType.DMA,
        ],
    )(jt, gt, jtail, gtail)

    n = job_idx.shape[0]
    n_per_w = n // NW
    rows_t = pltpu.VMEM((GRP, LINE), jnp.float32)
    idx_t = pltpu.VMEM((n_per_w,), jnp.int32)
    return pl.kernel(
        _cml_body,
        out_type=jax.ShapeDtypeStruct((n,), jnp.float32),
        mesh=mesh,
        compiler_params=compact,
        scratch_types=[
            idx_t, idx_t, idx_t, idx_t,
            rows_t, rows_t, rows_t, rows_t,
            pltpu.VMEM((n_per_w,), jnp.float32),
            pltpu.SemaphoreType.DMA,
            pltpu.SemaphoreType.DMA,
        ],
    )(job_idx, geek_idx, jp, gp)


def kernel(job, geek, job_emb_w, geek_emb_w):
    B, L = job.shape
    jf = job.astype(jnp.int32).reshape(-1)
    gf = geek.astype(jnp.int32).reshape(-1)
    out = _cml(jf, gf, job_emb_w, geek_emb_w)
    return out.reshape(B, L)
